# Initial kernel scaffold; baseline (speedup 1.0000x reference)
import jax
import jax.numpy as jnp
from jax.experimental import pallas as pl


def kernel(x, w_ih, w_hh, b_ih, b_hh, w_fc, b_fc):
    raise NotImplementedError("write your pallas kernel here")


# speedup vs baseline: 1.1407x; 1.1407x over previous
import jax
import jax.numpy as jnp
from jax.experimental import pallas as pl
from jax.experimental.pallas import tpu as pltpu

_H = 128
_CP = 128
_MAX_T = 16


def _lstm_fused_kernel(x_ref, wcat_ref, wfc_ref, bfc_ref, out_ref,
                       xpad_ref, h2_ref, c_ref):
    TB, In = x_ref.shape
    H = h2_ref.shape[1]
    BT = h2_ref.shape[0]
    T = TB // BT
    tc = pl.program_id(1)

    @pl.when(tc == 0)
    def _():
        h2_ref[...] = jnp.zeros_like(h2_ref)
        c_ref[...] = jnp.zeros_like(c_ref)
        lane = jax.lax.broadcasted_iota(jnp.int32, xpad_ref.shape, 1)
        xpad_ref[...] = jnp.where(lane == In, 1.0, 0.0).astype(jnp.float32)

    xpad_ref[:, 0:In] = x_ref[...]

    wcat = wcat_ref[...]

    def step(t, carry):
        h2, c = carry
        row = pl.multiple_of(t * BT, 8)
        lhs = jnp.concatenate([h2, xpad_ref[pl.ds(row, BT), :]], axis=1)
        gates = jnp.dot(lhs, wcat, preferred_element_type=jnp.float32)
        tg = jnp.tanh(gates)
        ti = tg[:, 0 * H:1 * H]
        tf = tg[:, 1 * H:2 * H]
        gg = tg[:, 2 * H:3 * H]
        to = tg[:, 3 * H:4 * H]
        c_new = 0.5 * ((c + tf * c) + (gg + ti * gg))
        tcn = jnp.tanh(c_new)
        h2_new = tcn + to * tcn
        return h2_new, c_new

    h2_f, c_f = jax.lax.fori_loop(0, T, step, (h2_ref[...], c_ref[...]),
                                  unroll=True)
    h2_ref[...] = h2_f
    c_ref[...] = c_f

    @pl.when(tc == pl.num_programs(1) - 1)
    def _():
        out_ref[...] = (jnp.dot(h2_f, wfc_ref[...],
                                preferred_element_type=jnp.float32)
                        + bfc_ref[...]).astype(out_ref.dtype)


def _time_chunk(S, max_chunk=_MAX_T):
    t = min(S, max_chunk)
    while S % t != 0:
        t -= 1
    return t


def kernel(x, w_ih, w_hh, b_ih, b_hh, w_fc, b_fc):
    B, S, In = x.shape
    H = w_hh.shape[1]
    C = w_fc.shape[0]
    CP = _CP

    B_pad = max(8, ((B + 7) // 8) * 8)
    NB = 2 if (B_pad >= 16 and B_pad % 16 == 0) else 1
    Bt = B_pad // NB

    T = _time_chunk(S)
    NT = S // T

    xf = x.astype(jnp.float32)
    if B_pad != B:
        xf = jnp.concatenate(
            [xf, jnp.zeros((B_pad - B, S, In), jnp.float32)], axis=0)

    x_blk = (jnp.transpose(xf, (1, 0, 2))
             .reshape(S, NB, Bt, In)
             .transpose(1, 0, 2, 3)
             .reshape(NB, S * Bt, In))

    col_scale = jnp.concatenate([
        jnp.full((2 * H,), 0.5, jnp.float32),
        jnp.ones((H,), jnp.float32),
        jnp.full((H,), 0.5, jnp.float32),
    ]).reshape(1, 4 * H)
    wcat = jnp.zeros((2 * H, 4 * H), jnp.float32)
    wcat = wcat.at[0:H, :].set(0.5 * w_hh.T.astype(jnp.float32))
    wcat = wcat.at[H:H + In, :].set(w_ih.T.astype(jnp.float32))
    wcat = wcat.at[H + In, :].set((b_ih + b_hh).astype(jnp.float32))
    wcat = wcat * col_scale

    wfc_pad = jnp.zeros((H, CP), jnp.float32).at[:, :C].set(
        0.5 * w_fc.T.astype(jnp.float32))
    bfc_pad = jnp.zeros((1, CP), jnp.float32).at[:, :C].set(
        b_fc.astype(jnp.float32).reshape(1, C))

    const = lambda b, t: (0, 0)
    out_pad = pl.pallas_call(
        _lstm_fused_kernel,
        out_shape=jax.ShapeDtypeStruct((B_pad, CP), jnp.float32),
        grid_spec=pltpu.PrefetchScalarGridSpec(
            num_scalar_prefetch=0,
            grid=(NB, NT),
            in_specs=[
                pl.BlockSpec((None, T * Bt, In), lambda b, t: (b, t, 0)),
                pl.BlockSpec((2 * H, 4 * H), const),
                pl.BlockSpec((H, CP), const),
                pl.BlockSpec((1, CP), const),
            ],
            out_specs=pl.BlockSpec((Bt, CP), lambda b, t: (b, 0)),
            scratch_shapes=[
                pltpu.VMEM((T * Bt, H), jnp.float32),
                pltpu.VMEM((Bt, H), jnp.float32),
                pltpu.VMEM((Bt, H), jnp.float32),
            ],
        ),
        compiler_params=pltpu.CompilerParams(
            dimension_semantics=("parallel", "arbitrary")),
    )(x_blk, wcat, wfc_pad, bfc_pad)
    return out_pad[:B, :C]


# speedup vs baseline: 9.1596x; 8.0298x over previous
import functools

import jax
import jax.numpy as jnp
from jax.experimental import pallas as pl
from jax.experimental.pallas import tpu as pltpu

_H = 128
_CP = 128


def _lstm_fused_kernel(x_ref, wcat_ref, wfc_ref, bfc_ref, out_ref,
                       xpad_ref, h2_ref, c_ref, *, T, Ip):
    H = h2_ref.shape[1]
    tc = pl.program_id(1)

    @pl.when(tc == 0)
    def _():
        h2_ref[...] = jnp.zeros_like(h2_ref)
        c_ref[...] = jnp.zeros_like(c_ref)
        xpad_ref[...] = jnp.zeros_like(xpad_ref)

    wcat = wcat_ref[...]

    def step(t, carry):
        h2, c = carry
        xpad_ref[:, 0:Ip] = x_ref[:, t * Ip:(t + 1) * Ip]
        lhs = jnp.concatenate([h2, xpad_ref[...]], axis=1)
        gates = jnp.dot(lhs, wcat, preferred_element_type=jnp.float32)
        tg = jnp.tanh(gates)
        ti = tg[:, 0 * H:1 * H]
        tf = tg[:, 1 * H:2 * H]
        gg = tg[:, 2 * H:3 * H]
        to = tg[:, 3 * H:4 * H]
        c_new = 0.5 * ((c + tf * c) + (gg + ti * gg))
        tcn = jnp.tanh(c_new)
        h2_new = tcn + to * tcn
        return h2_new, c_new

    carry = (h2_ref[...], c_ref[...])
    for t in range(T):
        carry = step(t, carry)
    h2_f, c_f = carry
    h2_ref[...] = h2_f
    c_ref[...] = c_f

    @pl.when(tc == pl.num_programs(1) - 1)
    def _():
        out_ref[...] = (jnp.dot(h2_f, wfc_ref[...],
                                preferred_element_type=jnp.float32)
                        + bfc_ref[...]).astype(out_ref.dtype)


def kernel(x, w_ih, w_hh, b_ih, b_hh, w_fc, b_fc):
    B, S, In = x.shape
    H = w_hh.shape[1]
    C = w_fc.shape[0]
    CP = _CP

    B_pad = max(8, ((B + 7) // 8) * 8)
    NB = 2 if (B_pad >= 16 and B_pad % 16 == 0) else 1
    Bt = B_pad // NB

    Ip = In + 1
    T = S
    for cand in range(S, 0, -1):
        if S % cand == 0 and (cand * Ip) % 128 == 0 and cand <= 32:
            T = cand
            break
    NT = S // T

    xf = x.astype(jnp.float32)
    if B_pad != B:
        xf = jnp.concatenate(
            [xf, jnp.zeros((B_pad - B, S, In), jnp.float32)], axis=0)

    x4 = jnp.concatenate([xf, jnp.ones((B_pad, S, 1), jnp.float32)], axis=2)
    x2d = x4.reshape(B_pad, S * Ip)

    col_scale = jnp.concatenate([
        jnp.full((2 * H,), 0.5, jnp.float32),
        jnp.ones((H,), jnp.float32),
        jnp.full((H,), 0.5, jnp.float32),
    ]).reshape(1, 4 * H)
    wcat = jnp.zeros((2 * H, 4 * H), jnp.float32)
    wcat = wcat.at[0:H, :].set(0.5 * w_hh.T.astype(jnp.float32))
    wcat = wcat.at[H:H + In, :].set(w_ih.T.astype(jnp.float32))
    wcat = wcat.at[H + In, :].set((b_ih + b_hh).astype(jnp.float32))
    wcat = wcat * col_scale

    wfc_pad = jnp.zeros((H, CP), jnp.float32).at[:, :C].set(
        0.5 * w_fc.T.astype(jnp.float32))
    bfc_pad = jnp.zeros((1, CP), jnp.float32).at[:, :C].set(
        b_fc.astype(jnp.float32).reshape(1, C))

    body = functools.partial(_lstm_fused_kernel, T=T, Ip=Ip)
    const = lambda b, t: (0, 0)
    out_pad = pl.pallas_call(
        body,
        out_shape=jax.ShapeDtypeStruct((B_pad, CP), jnp.float32),
        grid_spec=pltpu.PrefetchScalarGridSpec(
            num_scalar_prefetch=0,
            grid=(NB, NT),
            in_specs=[
                pl.BlockSpec((Bt, T * Ip), lambda b, t: (b, t)),
                pl.BlockSpec((2 * H, 4 * H), const),
                pl.BlockSpec((H, CP), const),
                pl.BlockSpec((1, CP), const),
            ],
            out_specs=pl.BlockSpec((Bt, CP), lambda b, t: (b, 0)),
            scratch_shapes=[
                pltpu.VMEM((Bt, H), jnp.float32),
                pltpu.VMEM((Bt, H), jnp.float32),
                pltpu.VMEM((Bt, H), jnp.float32),
            ],
        ),
        compiler_params=pltpu.CompilerParams(
            dimension_semantics=("parallel", "arbitrary")),
    )(x2d, wcat, wfc_pad, bfc_pad)
    return out_pad[:B, :C]


# speedup vs baseline: 9.5494x; 1.0426x over previous
import functools
import math

import jax
import jax.numpy as jnp
from jax.experimental import pallas as pl
from jax.experimental.pallas import tpu as pltpu

_H = 128
_CP = 128


def _lstm_fused_kernel(x_ref, wcat_ref, wfc_ref, bfc_ref, out_ref,
                       stage_ref, xpad_ref, h2_ref, c_ref, *, T, In, WPB):
    H = h2_ref.shape[1]
    W = T * In
    tc = pl.program_id(1)

    @pl.when(tc == 0)
    def _():
        h2_ref[...] = jnp.zeros_like(h2_ref)
        c_ref[...] = jnp.zeros_like(c_ref)
        lane = jax.lax.broadcasted_iota(jnp.int32, xpad_ref.shape, 1)
        xpad_ref[...] = jnp.where(lane == In, 1.0, 0.0).astype(jnp.float32)

    if WPB == 1:
        stage_ref[:, 0:W] = x_ref[...]
    else:
        w_idx = jax.lax.rem(tc, WPB)
        for j in range(WPB):
            @pl.when(w_idx == j)
            def _(j=j):
                stage_ref[:, 0:W] = x_ref[:, j * W:(j + 1) * W]

    wcat = wcat_ref[...]

    def step(t, carry):
        h2, c = carry
        xpad_ref[:, 0:In] = stage_ref[:, t * In:(t + 1) * In]
        lhs = jnp.concatenate([h2, xpad_ref[...]], axis=1)
        gates = jnp.dot(lhs, wcat, preferred_element_type=jnp.float32)
        tg = jnp.tanh(gates)
        ti = tg[:, 0 * H:1 * H]
        tf = tg[:, 1 * H:2 * H]
        gg = tg[:, 2 * H:3 * H]
        to = tg[:, 3 * H:4 * H]
        c_new = 0.5 * ((c + tf * c) + (gg + ti * gg))
        tcn = jnp.tanh(c_new)
        h2_new = tcn + to * tcn
        return h2_new, c_new

    carry = (h2_ref[...], c_ref[...])
    for t in range(T):
        carry = step(t, carry)
    h2_f, c_f = carry
    h2_ref[...] = h2_f
    c_ref[...] = c_f

    @pl.when(tc == pl.num_programs(1) - 1)
    def _():
        out_ref[...] = (jnp.dot(h2_f, wfc_ref[...],
                                preferred_element_type=jnp.float32)
                        + bfc_ref[...]).astype(out_ref.dtype)


def kernel(x, w_ih, w_hh, b_ih, b_hh, w_fc, b_fc):
    B, S, In = x.shape
    H = w_hh.shape[1]
    C = w_fc.shape[0]
    CP = _CP

    B_pad = max(8, ((B + 7) // 8) * 8)
    NB = 2 if (B_pad >= 16 and B_pad % 16 == 0) else 1
    Bt = B_pad // NB

    T = 1
    for cand in range(min(S, 32), 0, -1):
        if S % cand == 0:
            T = cand
            break
    NT = S // T
    W = T * In
    WPB = 128 // math.gcd(W, 128)
    if NT % WPB != 0:
        WPB = NT
    L = W * WPB

    xf = x.astype(jnp.float32)
    if B_pad != B:
        xf = jnp.concatenate(
            [xf, jnp.zeros((B_pad - B, S, In), jnp.float32)], axis=0)
    x2d = xf.reshape(B_pad, S * In)

    col_scale = jnp.concatenate([
        jnp.full((2 * H,), 0.5, jnp.float32),
        jnp.ones((H,), jnp.float32),
        jnp.full((H,), 0.5, jnp.float32),
    ]).reshape(1, 4 * H)
    wcat = jnp.zeros((2 * H, 4 * H), jnp.float32)
    wcat = wcat.at[0:H, :].set(0.5 * w_hh.T.astype(jnp.float32))
    wcat = wcat.at[H:H + In, :].set(w_ih.T.astype(jnp.float32))
    wcat = wcat.at[H + In, :].set((b_ih + b_hh).astype(jnp.float32))
    wcat = wcat * col_scale

    wfc_pad = jnp.zeros((H, CP), jnp.float32).at[:, :C].set(
        0.5 * w_fc.T.astype(jnp.float32))
    bfc_pad = jnp.zeros((1, CP), jnp.float32).at[:, :C].set(
        b_fc.astype(jnp.float32).reshape(1, C))

    body = functools.partial(_lstm_fused_kernel, T=T, In=In, WPB=WPB)
    const = lambda b, t: (0, 0)
    out_pad = pl.pallas_call(
        body,
        out_shape=jax.ShapeDtypeStruct((B_pad, CP), jnp.float32),
        grid_spec=pltpu.PrefetchScalarGridSpec(
            num_scalar_prefetch=0,
            grid=(NB, NT),
            in_specs=[
                pl.BlockSpec((Bt, L), lambda b, t: (b, t // WPB)),
                pl.BlockSpec((2 * H, 4 * H), const),
                pl.BlockSpec((H, CP), const),
                pl.BlockSpec((1, CP), const),
            ],
            out_specs=pl.BlockSpec((Bt, CP), lambda b, t: (b, 0)),
            scratch_shapes=[
                pltpu.VMEM((Bt, ((W + 127) // 128) * 128),
                           jnp.float32),
                pltpu.VMEM((Bt, H), jnp.float32),
                pltpu.VMEM((Bt, H), jnp.float32),
                pltpu.VMEM((Bt, H), jnp.float32),
            ],
        ),
        compiler_params=pltpu.CompilerParams(
            dimension_semantics=("parallel", "arbitrary")),
    )(x2d, wcat, wfc_pad, bfc_pad)
    return out_pad[:B, :C]
